# baseline (speedup 1.0000x reference)
import jax
import jax.numpy as jnp
from jax import lax
from jax.experimental import pallas as pl
from jax.experimental.pallas import tpu as pltpu
from jax.experimental.pallas import tpu_sc as plsc

N = 32768
B = 16
D = 768
L = 16
NC, NS = 2, 16
NW = NC * NS
ROWS_W = N // NW
CH = 32
NCH = ROWS_W // CH
NV = D // L


def _sc_body(xf_hbm, batch_hbm, w_hbm, bvec_hbm,
             wout_hbm, part_hbm, cnt_hbm,
             xb0, xb1, batchb, wv, bv, woutb, accb, cntb,
             sem0, sem1):
    c = lax.axis_index("c")
    s = lax.axis_index("s")
    wid = s * NC + c
    row0 = wid * ROWS_W

    pltpu.sync_copy(w_hbm, wv)
    pltpu.sync_copy(bvec_hbm, bv)
    pltpu.sync_copy(batch_hbm.at[pl.ds(row0, ROWS_W)], batchb)

    zero16 = jnp.zeros((L,), jnp.float32)

    def zb_(j, carry):
        accb[pl.ds(j * L, L)] = zero16
        return carry

    lax.fori_loop(0, B * D // L, zb_, 0)

    pltpu.async_copy(xf_hbm.at[pl.ds(row0 * D, CH * D)], xb0, sem0)

    iot = lax.broadcasted_iota(jnp.int32, (L,), 0)

    def lane_gather(v, idx):
        return lax.gather(
            v, idx.reshape(L, 1),
            lax.GatherDimensionNumbers(
                offset_dims=(), collapsed_slice_dims=(0,), start_index_map=(0,)),
            (1,), mode=lax.GatherScatterMode.PROMISE_IN_BOUNDS)

    def process(xb, crb, cnt_vec):
        def tile_body(t, cnt_vec):
            tb = t * L

            def row_dot(r, zvec):
                def dj(j, a):
                    return a + xb[pl.ds((tb + r) * D + j * L, L)] * wv[pl.ds(j * L, L)]
                accv = lax.fori_loop(0, NV, dj, zero16)
                for sh in (8, 4, 2, 1):
                    accv = accv + lane_gather(accv, (iot + sh) & (L - 1))
                return jnp.where(iot == r, accv, zvec)

            zvec = lax.fori_loop(0, L, row_dot, zero16)
            zb = zvec + bv[...]
            w16 = 1.0 / (1.0 + jnp.exp(-zb))
            cnt_vec = cnt_vec + jnp.where(zb >= 0.0, 1.0, 0.0)
            woutb[pl.ds(crb + tb, L)] = w16

            def row_scale(r, carry):
                ridx = jnp.full((L,), crb + tb + r, jnp.int32)
                w_b = plsc.load_gather(woutb, [ridx])
                seg_b = plsc.load_gather(batchb, [ridx])
                base = seg_b * D + iot

                def sj(j, carry2):
                    v = xb[pl.ds((tb + r) * D + j * L, L)] * w_b
                    plsc.addupdate_scatter(accb, [base + j * L], v)
                    return carry2

                return lax.fori_loop(0, NV, sj, carry)

            lax.fori_loop(0, L, row_scale, 0)
            return cnt_vec

        return lax.fori_loop(0, CH // L, tile_body, cnt_vec)

    def pair(i, cnt_vec):
        off1 = (2 * i + 1) * CH
        pltpu.async_copy(xf_hbm.at[pl.ds((row0 + off1) * D, CH * D)], xb1, sem1)
        pltpu.make_async_copy(xf_hbm.at[pl.ds(0, CH * D)], xb0, sem0).wait()
        cnt_vec = process(xb0, (2 * i) * CH, cnt_vec)

        @pl.when(i < NCH // 2 - 1)
        def _():
            off2 = (2 * i + 2) * CH
            pltpu.async_copy(xf_hbm.at[pl.ds((row0 + off2) * D, CH * D)], xb0, sem0)

        pltpu.make_async_copy(xf_hbm.at[pl.ds(0, CH * D)], xb1, sem1).wait()
        cnt_vec = process(xb1, off1, cnt_vec)
        return cnt_vec

    cnt_vec = lax.fori_loop(0, NCH // 2, pair, zero16)

    cntb[...] = cnt_vec
    pltpu.sync_copy(woutb, wout_hbm.at[pl.ds(row0, ROWS_W)])
    pltpu.sync_copy(accb, part_hbm.at[wid])
    pltpu.sync_copy(cntb, cnt_hbm.at[wid])


def _combine_body(part_ref, cnt_ref, pooled_ref, ratio_ref):
    p = part_ref[...]
    pooled_ref[...] = jnp.sum(p, axis=0)
    ratio_ref[...] = jnp.sum(cnt_ref[...]).reshape(1, 1) * (1.0 / N)


def kernel(x, batch, ptr, W, b):
    del ptr
    xf = x.reshape(-1)
    wf = W.reshape(-1)
    bvec = jnp.broadcast_to(b, (L,))

    mesh = plsc.VectorSubcoreMesh(core_axis_name="c", subcore_axis_name="s",
                                  num_cores=NC, num_subcores=NS)
    wflat, part, cnt = pl.kernel(
        _sc_body,
        out_type=[
            jax.ShapeDtypeStruct((N,), jnp.float32),
            jax.ShapeDtypeStruct((NW, B * D), jnp.float32),
            jax.ShapeDtypeStruct((NW, L), jnp.float32),
        ],
        mesh=mesh,
        compiler_params=pltpu.CompilerParams(needs_layout_passes=False),
        scratch_types=[
            pltpu.VMEM((CH * D,), jnp.float32),
            pltpu.VMEM((CH * D,), jnp.float32),
            pltpu.VMEM((ROWS_W,), jnp.int32),
            pltpu.VMEM((D,), jnp.float32),
            pltpu.VMEM((L,), jnp.float32),
            pltpu.VMEM((ROWS_W,), jnp.float32),
            pltpu.VMEM((B * D,), jnp.float32),
            pltpu.VMEM((L,), jnp.float32),
            pltpu.SemaphoreType.DMA,
            pltpu.SemaphoreType.DMA,
        ],
    )(xf, batch, wf, bvec)

    pooled, ratio = pl.pallas_call(
        _combine_body,
        out_shape=[
            jax.ShapeDtypeStruct((B, D), jnp.float32),
            jax.ShapeDtypeStruct((1, 1), jnp.float32),
        ],
    )(part.reshape(NW, B, D), cnt)

    return pooled, wflat.reshape(N, 1), ratio.reshape(())


# speedup vs baseline: 1.3503x; 1.3503x over previous
import jax
import jax.numpy as jnp
from jax import lax
from jax.experimental import pallas as pl
from jax.experimental.pallas import tpu as pltpu
from jax.experimental.pallas import tpu_sc as plsc

N = 32768
B = 16
D = 768
L = 16
NC, NS = 2, 16
NW = NC * NS
ROWS_W = N // NW
CH = 32
NCH = ROWS_W // CH
NV = D // L


def _sc_body(xf_hbm, batch_hbm, w_hbm, bvec_hbm,
             wout_hbm, part_hbm, cnt_hbm,
             xb0, xb1, batchb, wv, bv, woutb, accb, cntb,
             sem0, sem1):
    c = lax.axis_index("c")
    s = lax.axis_index("s")
    wid = s * NC + c
    row0 = wid * ROWS_W

    pltpu.sync_copy(w_hbm, wv)
    pltpu.sync_copy(bvec_hbm, bv)
    pltpu.sync_copy(batch_hbm.at[pl.ds(row0, ROWS_W)], batchb)

    zero16 = jnp.zeros((L,), jnp.float32)

    def zb_(j, carry):
        accb[pl.ds(j * L, L)] = zero16
        return carry

    lax.fori_loop(0, B * D // L, zb_, 0)

    pltpu.async_copy(xf_hbm.at[pl.ds(row0 * D, CH * D)], xb0, sem0)

    iot = lax.broadcasted_iota(jnp.int32, (L,), 0)

    def lane_gather(v, idx):
        return lax.gather(
            v, idx.reshape(L, 1),
            lax.GatherDimensionNumbers(
                offset_dims=(), collapsed_slice_dims=(0,), start_index_map=(0,)),
            (1,), mode=lax.GatherScatterMode.PROMISE_IN_BOUNDS)

    def process(xb, crb, cnt_vec):
        def row_body(r, cnt_vec):
            rowoff = r * D
            accv = zero16
            for j in range(NV):
                accv = accv + xb[pl.ds(rowoff + j * L, L)] * wv[pl.ds(j * L, L)]
            for sh in (8, 4, 2, 1):
                accv = accv + lane_gather(accv, (iot + sh) & (L - 1))
            zb = accv + bv[...]
            w_row = 1.0 / (1.0 + jnp.exp(-zb))
            cnt_vec = cnt_vec + jnp.where(zb >= 0.0, 1.0 / L, 0.0)
            rid = jnp.full((L,), crb + r, jnp.int32)
            plsc.store_scatter(woutb, [rid], w_row, mask=iot == 0)
            seg_b = plsc.load_gather(batchb, [rid])
            base = seg_b * D + iot
            for j in range(NV):
                v = xb[pl.ds(rowoff + j * L, L)] * w_row
                plsc.addupdate_scatter(accb, [base + j * L], v)
            return cnt_vec

        return lax.fori_loop(0, CH, row_body, cnt_vec)

    def pair(i, cnt_vec):
        off1 = (2 * i + 1) * CH
        pltpu.async_copy(xf_hbm.at[pl.ds((row0 + off1) * D, CH * D)], xb1, sem1)
        pltpu.make_async_copy(xf_hbm.at[pl.ds(0, CH * D)], xb0, sem0).wait()
        cnt_vec = process(xb0, (2 * i) * CH, cnt_vec)

        @pl.when(i < NCH // 2 - 1)
        def _():
            off2 = (2 * i + 2) * CH
            pltpu.async_copy(xf_hbm.at[pl.ds((row0 + off2) * D, CH * D)], xb0, sem0)

        pltpu.make_async_copy(xf_hbm.at[pl.ds(0, CH * D)], xb1, sem1).wait()
        cnt_vec = process(xb1, off1, cnt_vec)
        return cnt_vec

    cnt_vec = lax.fori_loop(0, NCH // 2, pair, zero16)

    cntb[...] = cnt_vec
    pltpu.sync_copy(woutb, wout_hbm.at[pl.ds(row0, ROWS_W)])
    pltpu.sync_copy(accb, part_hbm.at[wid])
    pltpu.sync_copy(cntb, cnt_hbm.at[wid])


def _combine_body(part_ref, cnt_ref, pooled_ref, ratio_ref):
    p = part_ref[...]
    pooled_ref[...] = jnp.sum(p, axis=0)
    ratio_ref[...] = jnp.sum(cnt_ref[...]).reshape(1, 1) * (1.0 / N)


def kernel(x, batch, ptr, W, b):
    del ptr
    xf = x.reshape(-1)
    wf = W.reshape(-1)
    bvec = jnp.broadcast_to(b, (L,))

    mesh = plsc.VectorSubcoreMesh(core_axis_name="c", subcore_axis_name="s",
                                  num_cores=NC, num_subcores=NS)
    wflat, part, cnt = pl.kernel(
        _sc_body,
        out_type=[
            jax.ShapeDtypeStruct((N,), jnp.float32),
            jax.ShapeDtypeStruct((NW, B * D), jnp.float32),
            jax.ShapeDtypeStruct((NW, L), jnp.float32),
        ],
        mesh=mesh,
        compiler_params=pltpu.CompilerParams(needs_layout_passes=False),
        scratch_types=[
            pltpu.VMEM((CH * D,), jnp.float32),
            pltpu.VMEM((CH * D,), jnp.float32),
            pltpu.VMEM((ROWS_W,), jnp.int32),
            pltpu.VMEM((D,), jnp.float32),
            pltpu.VMEM((L,), jnp.float32),
            pltpu.VMEM((ROWS_W,), jnp.float32),
            pltpu.VMEM((B * D,), jnp.float32),
            pltpu.VMEM((L,), jnp.float32),
            pltpu.SemaphoreType.DMA,
            pltpu.SemaphoreType.DMA,
        ],
    )(xf, batch, wf, bvec)

    pooled, ratio = pl.pallas_call(
        _combine_body,
        out_shape=[
            jax.ShapeDtypeStruct((B, D), jnp.float32),
            jax.ShapeDtypeStruct((1, 1), jnp.float32),
        ],
    )(part.reshape(NW, B, D), cnt)

    return pooled, wflat.reshape(N, 1), ratio.reshape(())
